# Initial kernel scaffold; baseline (speedup 1.0000x reference)
import jax
import jax.numpy as jnp
from jax.experimental import pallas as pl


def kernel(h, x, edges, v, c_edge_index, edge_attr, params):
    raise NotImplementedError("write your pallas kernel here")


# speedup vs baseline: 3.6361x; 3.6361x over previous
import functools

import jax
import jax.numpy as jnp
from jax import lax
from jax.experimental import pallas as pl
from jax.experimental.pallas import tpu as pltpu, tpu_sc as plsc

F32 = jnp.float32
I32 = jnp.int32

N = 10000
E = 320000
EC = 5000
HF = 128
DE = 16
NL = 4

NC, NS = 2, 16
NW = NC * NS
EPW = E // NW
CHG = 80
NCHG = EPW // CHG
GPC = CHG // 16
CHS = 200
NCHS = EPW // CHS
ECP = 5120
ECPW = ECP // NW
NPAD = 10240
RPT = NPAD // NS

BN = 2000
BE = 2000


def _silu(t):
    return t * jax.nn.sigmoid(t)


def _bspec(bn, width):
    return pl.BlockSpec((bn, width), lambda i: (i, 0))


def _wspec(shape):
    return pl.BlockSpec(shape, lambda i: (0, 0))


def _emb_body(h_ref, w_ref, b_ref, o_ref):
    o_ref[...] = jnp.dot(h_ref[...], w_ref[...],
                         preferred_element_type=F32) + b_ref[...]


_emb = pl.pallas_call(
    _emb_body,
    grid=(N // BN,),
    in_specs=[_bspec(BN, HF), _wspec((HF, HF)), _wspec((1, HF))],
    out_specs=_bspec(BN, HF),
    out_shape=jax.ShapeDtypeStruct((N, HF), F32),
)


def _pre_body(h_ref, wr, wc, wcr, wcc, wv1, bv1, wv2, bv2,
              a_o, b_o, ac_o, bc_o, vs_o):
    hh = h_ref[...]
    a_o[...] = jnp.dot(hh, wr[...], preferred_element_type=F32)
    b_o[...] = jnp.dot(hh, wc[...], preferred_element_type=F32)
    ac_o[...] = jnp.dot(hh, wcr[...], preferred_element_type=F32)
    bc_o[...] = jnp.dot(hh, wcc[...], preferred_element_type=F32)
    t = _silu(jnp.dot(hh, wv1[...], preferred_element_type=F32) + bv1[...])
    vs_o[...] = jnp.dot(t, wv2[...], preferred_element_type=F32) + bv2[...]


_pre = pl.pallas_call(
    _pre_body,
    grid=(N // BN,),
    in_specs=[_bspec(BN, HF), _wspec((HF, HF)), _wspec((HF, HF)),
              _wspec((HF, HF)), _wspec((HF, HF)), _wspec((HF, HF)),
              _wspec((1, HF)), _wspec((HF, 1)), _wspec((1, 1))],
    out_specs=[_bspec(BN, HF), _bspec(BN, HF), _bspec(BN, HF),
               _bspec(BN, HF), _bspec(BN, 1)],
    out_shape=[jax.ShapeDtypeStruct((N, HF), F32),
               jax.ShapeDtypeStruct((N, HF), F32),
               jax.ShapeDtypeStruct((N, HF), F32),
               jax.ShapeDtypeStruct((N, HF), F32),
               jax.ShapeDtypeStruct((N, 1), F32)],
)


def _edge_core(ga, gb, xd, ea, wea, wd2, be1, we2, be2, wx1, bx1, wx2, bx2):
    xdv = xd[...]
    d2 = xdv[:, 3:4]
    col = lax.broadcasted_iota(I32, xdv.shape, 1)
    diff = jnp.where(col < 3, xdv, 0.0)
    pre = (ga[...] + gb[...] + d2 * wd2[...]
           + jnp.dot(ea[...], wea[...], preferred_element_type=F32)
           + be1[...])
    m = _silu(jnp.dot(_silu(pre), we2[...],
                      preferred_element_type=F32) + be2[...])
    u = _silu(jnp.dot(m, wx1[...], preferred_element_type=F32) + bx1[...])
    coef = jnp.dot(u, wx2[...], preferred_element_type=F32) + bx2[...]
    return m, diff * coef


def _edge_body_full(ga, gb, xd, ea, wea, wd2, be1, we2, be2,
                    wx1, bx1, wx2, bx2, m_o, dc_o):
    m, dc = _edge_core(ga, gb, xd, ea, wea, wd2, be1, we2, be2,
                       wx1, bx1, wx2, bx2)
    m_o[...] = m
    dc_o[:, 0:16] = dc
    dc_o[:, 16:HF] = jnp.zeros((dc.shape[0], HF - 16), F32)


def _edge_body_last(ga, gb, xd, ea, wea, wd2, be1, we2, be2,
                    wx1, bx1, wx2, bx2, dc_o):
    _, dc = _edge_core(ga, gb, xd, ea, wea, wd2, be1, we2, be2,
                       wx1, bx1, wx2, bx2)
    dc_o[:, 0:16] = dc
    dc_o[:, 16:HF] = jnp.zeros((dc.shape[0], HF - 16), F32)


_EDGE_IN_SPECS = [
    _bspec(BE, HF), _bspec(BE, HF), _bspec(BE, 16),
    _bspec(BE, DE), _wspec((DE, HF)), _wspec((1, HF)), _wspec((1, HF)),
    _wspec((HF, HF)), _wspec((1, HF)), _wspec((HF, HF)), _wspec((1, HF)),
    _wspec((HF, 1)), _wspec((1, 1)),
]

_edge_full = pl.pallas_call(
    _edge_body_full,
    grid=(E // BE,),
    in_specs=_EDGE_IN_SPECS,
    out_specs=[_bspec(BE, HF), _bspec(BE, HF)],
    out_shape=[jax.ShapeDtypeStruct((E, HF), F32),
               jax.ShapeDtypeStruct((E, HF), F32)],
)

_edge_last = pl.pallas_call(
    _edge_body_last,
    grid=(E // BE,),
    in_specs=_EDGE_IN_SPECS,
    out_specs=_bspec(BE, HF),
    out_shape=jax.ShapeDtypeStruct((E, HF), F32),
)


def _cmlp_body(gac, gbc, xd, wd2c, bc1, wc2, bc2, dcc_o):
    xdv = xd[...]
    d2 = xdv[:, 3:4]
    col = lax.broadcasted_iota(I32, xdv.shape, 1)
    diff = jnp.where(col < 3, xdv, 0.0)
    pre = gac[...] + gbc[...] + d2 * wd2c[...] + bc1[...]
    cc = jnp.dot(_silu(pre), wc2[...], preferred_element_type=F32) + bc2[...]
    dcc_o[:, 0:16] = diff * cc
    dcc_o[:, 16:HF] = jnp.zeros((xdv.shape[0], HF - 16), F32)


_cmlp = pl.pallas_call(
    _cmlp_body,
    grid=(1,),
    in_specs=[_bspec(ECP, HF), _bspec(ECP, HF), _bspec(ECP, 16),
              _wspec((1, HF)), _wspec((1, HF)), _wspec((HF, 1)),
              _wspec((1, 1))],
    out_specs=_bspec(ECP, HF),
    out_shape=jax.ShapeDtypeStruct((ECP, HF), F32),
)


def _upd_full_body(h, p0m, p1m, d0, d1, x16, v16, vs,
                   wh1a, wh1b, bh1, wh2, bh2, h_o, x_o, v_o):
    aggh = p0m[...] + p1m[...]
    aggx = d0[...] + d1[...]
    vn = vs[...] * v16[...] + aggx
    v_o[...] = vn
    x_o[...] = x16[...] + vn
    q = _silu(jnp.dot(h[...], wh1a[...], preferred_element_type=F32)
              + jnp.dot(aggh, wh1b[...], preferred_element_type=F32)
              + bh1[...])
    h_o[...] = h[...] + jnp.dot(q, wh2[...],
                                preferred_element_type=F32) + bh2[...]


_upd_full = pl.pallas_call(
    _upd_full_body,
    grid=(N // BN,),
    in_specs=[_bspec(BN, HF), _bspec(BN, HF), _bspec(BN, HF),
              _bspec(BN, 16), _bspec(BN, 16), _bspec(BN, 16),
              _bspec(BN, 16), _bspec(BN, 1), _wspec((HF, HF)),
              _wspec((HF, HF)), _wspec((1, HF)), _wspec((HF, HF)),
              _wspec((1, HF))],
    out_specs=[_bspec(BN, HF), _bspec(BN, 16), _bspec(BN, 16)],
    out_shape=[jax.ShapeDtypeStruct((N, HF), F32),
               jax.ShapeDtypeStruct((N, 16), F32),
               jax.ShapeDtypeStruct((N, 16), F32)],
)


def _upd_light_body(d0, d1, x16, v16, vs, x_o, v_o):
    aggx = d0[...] + d1[...]
    vn = vs[...] * v16[...] + aggx
    v_o[...] = vn
    x_o[...] = x16[...] + vn


_upd_light = pl.pallas_call(
    _upd_light_body,
    grid=(N // BN,),
    in_specs=[_bspec(BN, 16), _bspec(BN, 16), _bspec(BN, 16),
              _bspec(BN, 16), _bspec(BN, 1)],
    out_specs=[_bspec(BN, 16), _bspec(BN, 16)],
    out_shape=[jax.ShapeDtypeStruct((N, 16), F32),
               jax.ShapeDtypeStruct((N, 16), F32)],
)


def _sc_mesh():
    return plsc.VectorSubcoreMesh(core_axis_name="c", subcore_axis_name="s")


def _diff_groups(idx_r, idx_c, xx, xy, xz, buf_xd, ngroups):
    iota16 = lax.iota(I32, 16)
    c0 = jnp.zeros((16,), I32)
    for j in range(ngroups):
        ir = idx_r[pl.ds(j * 16, 16)]
        ic = idx_c[pl.ds(j * 16, 16)]
        dx = plsc.load_gather(xx, [ir]) - plsc.load_gather(xx, [ic])
        dy = plsc.load_gather(xy, [ir]) - plsc.load_gather(xy, [ic])
        dz = plsc.load_gather(xz, [ir]) - plsc.load_gather(xz, [ic])
        d2 = dx * dx + dy * dy + dz * dz
        rj = iota16 + (j * 16)
        plsc.store_scatter(buf_xd, [rj, c0], dx)
        plsc.store_scatter(buf_xd, [rj, c0 + 1], dy)
        plsc.store_scatter(buf_xd, [rj, c0 + 2], dz)
        plsc.store_scatter(buf_xd, [rj, c0 + 3], d2)


def _zero_rows(buf, nrows):
    z = jnp.zeros((16,), F32)

    def zbody(r, carry):
        buf[r] = z
        return carry

    lax.fori_loop(0, nrows, zbody, 0)


@functools.partial(
    pl.kernel,
    out_type=[jax.ShapeDtypeStruct((E, HF), F32),
              jax.ShapeDtypeStruct((E, HF), F32),
              jax.ShapeDtypeStruct((E, 16), F32)],
    mesh=_sc_mesh(),
    compiler_params=pltpu.CompilerParams(needs_layout_passes=False),
    scratch_types=[pltpu.VMEM((CHG,), I32),
                   pltpu.VMEM((CHG,), I32),
                   pltpu.VMEM((CHG, HF), F32),
                   pltpu.VMEM((CHG, HF), F32),
                   pltpu.VMEM((CHG, 16), F32),
                   pltpu.VMEM((N,), F32),
                   pltpu.VMEM((N,), F32),
                   pltpu.VMEM((N,), F32),
                   pltpu.SemaphoreType.DMA,
                   pltpu.SemaphoreType.DMA],
)
def _gather_main(a_hbm, b_hbm, xx_hbm, xy_hbm, xz_hbm, row_hbm, col_hbm,
                 ga_hbm, gb_hbm, xd_hbm,
                 idx_r, idx_c, buf_a, buf_b, buf_xd, xx, xy, xz, s0, s1):
    wid = lax.axis_index("s") * NC + lax.axis_index("c")
    base0 = wid * EPW
    pltpu.sync_copy(xx_hbm, xx)
    pltpu.sync_copy(xy_hbm, xy)
    pltpu.sync_copy(xz_hbm, xz)
    _zero_rows(buf_xd, CHG)

    def body(i, carry):
        base = base0 + i * CHG
        pltpu.sync_copy(row_hbm.at[pl.ds(base, CHG)], idx_r)
        pltpu.sync_copy(col_hbm.at[pl.ds(base, CHG)], idx_c)
        ca = pltpu.async_copy(a_hbm.at[idx_r], buf_a, s0)
        cb = pltpu.async_copy(b_hbm.at[idx_c], buf_b, s1)
        _diff_groups(idx_r, idx_c, xx, xy, xz, buf_xd, GPC)
        ca.wait()
        cb.wait()
        pltpu.sync_copy(buf_a, ga_hbm.at[pl.ds(base, CHG)])
        pltpu.sync_copy(buf_b, gb_hbm.at[pl.ds(base, CHG)])
        pltpu.sync_copy(buf_xd, xd_hbm.at[pl.ds(base, CHG)])
        return carry

    lax.fori_loop(0, NCHG, body, 0)


@functools.partial(
    pl.kernel,
    out_type=[jax.ShapeDtypeStruct((ECP, HF), F32),
              jax.ShapeDtypeStruct((ECP, HF), F32),
              jax.ShapeDtypeStruct((ECP, 16), F32)],
    mesh=_sc_mesh(),
    compiler_params=pltpu.CompilerParams(needs_layout_passes=False),
    scratch_types=[pltpu.VMEM((ECPW,), I32),
                   pltpu.VMEM((ECPW,), I32),
                   pltpu.VMEM((ECPW, HF), F32),
                   pltpu.VMEM((ECPW, HF), F32),
                   pltpu.VMEM((ECPW, 16), F32),
                   pltpu.VMEM((N,), F32),
                   pltpu.VMEM((N,), F32),
                   pltpu.VMEM((N,), F32),
                   pltpu.SemaphoreType.DMA,
                   pltpu.SemaphoreType.DMA],
)
def _gather_c(a_hbm, b_hbm, xx_hbm, xy_hbm, xz_hbm, row_hbm, col_hbm,
              ga_hbm, gb_hbm, xd_hbm,
              idx_r, idx_c, buf_a, buf_b, buf_xd, xx, xy, xz, s0, s1):
    wid = lax.axis_index("s") * NC + lax.axis_index("c")
    base = wid * ECPW
    pltpu.sync_copy(xx_hbm, xx)
    pltpu.sync_copy(xy_hbm, xy)
    pltpu.sync_copy(xz_hbm, xz)
    _zero_rows(buf_xd, ECPW)
    pltpu.sync_copy(row_hbm.at[pl.ds(base, ECPW)], idx_r)
    pltpu.sync_copy(col_hbm.at[pl.ds(base, ECPW)], idx_c)
    ca = pltpu.async_copy(a_hbm.at[idx_r], buf_a, s0)
    cb = pltpu.async_copy(b_hbm.at[idx_c], buf_b, s1)
    _diff_groups(idx_r, idx_c, xx, xy, xz, buf_xd, ECPW // 16)
    ca.wait()
    cb.wait()
    pltpu.sync_copy(buf_a, ga_hbm.at[pl.ds(base, ECPW)])
    pltpu.sync_copy(buf_b, gb_hbm.at[pl.ds(base, ECPW)])
    pltpu.sync_copy(buf_xd, xd_hbm.at[pl.ds(base, ECPW)])


@functools.partial(
    pl.kernel,
    out_type=[jax.ShapeDtypeStruct((NPAD, HF), F32),
              jax.ShapeDtypeStruct((NPAD, HF), F32)],
    mesh=_sc_mesh(),
    compiler_params=pltpu.CompilerParams(needs_layout_passes=False),
    scratch_types=[pltpu.VMEM((CHS,), I32),
                   pltpu.VMEM((CHS, HF), F32),
                   pltpu.VMEM_SHARED((NPAD, HF), F32)],
)
def _scatter_m(val_hbm, idx_hbm, zero_hbm, out0, out1, idx_v, buf, table):
    c = lax.axis_index("c")
    s = lax.axis_index("s")
    pltpu.sync_copy(zero_hbm, table.at[pl.ds(s * RPT, RPT)])
    plsc.subcore_barrier()
    base0 = (c * NS + s) * EPW

    def body(i, carry):
        base = base0 + i * CHS
        pltpu.sync_copy(idx_hbm.at[pl.ds(base, CHS)], idx_v)
        pltpu.sync_copy(val_hbm.at[pl.ds(base, CHS)], buf)
        pltpu.sync_copy(buf, table.at[idx_v], add=True)
        return carry

    lax.fori_loop(0, NCHS, body, 0)
    plsc.subcore_barrier()

    @pl.when(c == 0)
    def _():
        pltpu.sync_copy(table.at[pl.ds(s * RPT, RPT)],
                        out0.at[pl.ds(s * RPT, RPT)])

    @pl.when(c == 1)
    def _():
        pltpu.sync_copy(table.at[pl.ds(s * RPT, RPT)],
                        out1.at[pl.ds(s * RPT, RPT)])


@functools.partial(
    pl.kernel,
    out_type=[jax.ShapeDtypeStruct((NPAD, HF), F32),
              jax.ShapeDtypeStruct((NPAD, HF), F32)],
    mesh=_sc_mesh(),
    compiler_params=pltpu.CompilerParams(needs_layout_passes=False),
    scratch_types=[pltpu.VMEM((CHS,), I32),
                   pltpu.VMEM((CHS, HF), F32),
                   pltpu.VMEM((ECPW,), I32),
                   pltpu.VMEM((ECPW, HF), F32),
                   pltpu.VMEM_SHARED((NPAD, HF), F32)],
)
def _scatter_xc(val_hbm, idx_hbm, cval_hbm, cidx_hbm, zero_hbm, out0, out1,
                idx_v, buf, cidx_v, cbuf, table):
    c = lax.axis_index("c")
    s = lax.axis_index("s")
    pltpu.sync_copy(zero_hbm, table.at[pl.ds(s * RPT, RPT)])
    plsc.subcore_barrier()
    base0 = (c * NS + s) * EPW

    def body(i, carry):
        base = base0 + i * CHS
        pltpu.sync_copy(idx_hbm.at[pl.ds(base, CHS)], idx_v)
        pltpu.sync_copy(val_hbm.at[pl.ds(base, CHS)], buf)
        pltpu.sync_copy(buf, table.at[idx_v], add=True)
        return carry

    lax.fori_loop(0, NCHS, body, 0)
    cbase = (c * NS + s) * ECPW
    pltpu.sync_copy(cidx_hbm.at[pl.ds(cbase, ECPW)], cidx_v)
    pltpu.sync_copy(cval_hbm.at[pl.ds(cbase, ECPW)], cbuf)
    pltpu.sync_copy(cbuf, table.at[cidx_v], add=True)
    plsc.subcore_barrier()

    @pl.when(c == 0)
    def _():
        pltpu.sync_copy(table.at[pl.ds(s * RPT, RPT)],
                        out0.at[pl.ds(s * RPT, RPT)])

    @pl.when(c == 1)
    def _():
        pltpu.sync_copy(table.at[pl.ds(s * RPT, RPT)],
                        out1.at[pl.ds(s * RPT, RPT)])


def kernel(h, x, edges, v, c_edge_index, edge_attr, params):
    row = edges[0]
    col = edges[1]
    crow = jnp.pad(c_edge_index[0], (0, ECP - EC))
    ccol = jnp.pad(c_edge_index[1], (0, ECP - EC))
    x16 = jnp.pad(x, ((0, 0), (0, 13)))
    v16 = jnp.pad(v, ((0, 0), (0, 13)))
    zero128 = jnp.zeros((RPT, HF), F32)

    hcur = _emb(h, params["W_emb"], params["b_emb"].reshape(1, HF))
    for li, p in enumerate(params["layers"]):
        last = li == NL - 1
        We1 = p["We1"]
        wr, wc = We1[0:HF], We1[HF:2 * HF]
        wd2, wea = We1[2 * HF:2 * HF + 1], We1[2 * HF + 1:]
        Wc1 = p["Wc1"]
        wcr, wcc, wd2c = Wc1[0:HF], Wc1[HF:2 * HF], Wc1[2 * HF:2 * HF + 1]

        a, b, ac, bc, vs = _pre(
            hcur, wr, wc, wcr, wcc, p["Wv1"], p["bv1"].reshape(1, HF),
            p["Wv2"], p["bv2"].reshape(1, 1))
        xxc, xyc, xzc = x16[:, 0], x16[:, 1], x16[:, 2]
        ga, gb, xd = _gather_main(a, b, xxc, xyc, xzc, row, col)
        gac, gbc, xcd = _gather_c(ac, bc, xxc, xyc, xzc, crow, ccol)
        dcc = _cmlp(gac, gbc, xcd, wd2c, p["bc1"].reshape(1, HF),
                    p["Wc2"], p["bc2"].reshape(1, 1))
        eargs = (ga, gb, xd, edge_attr, wea, wd2,
                 p["be1"].reshape(1, HF), p["We2"], p["be2"].reshape(1, HF),
                 p["Wx1"], p["bx1"].reshape(1, HF), p["Wx2"],
                 p["bx2"].reshape(1, 1))
        if last:
            dc = _edge_last(*eargs)
            d0, d1 = _scatter_xc(dc, row, dcc, crow, zero128)
            x16, v16 = _upd_light(d0[:N, :16], d1[:N, :16], x16, v16, vs)
        else:
            m, dc = _edge_full(*eargs)
            s0, s1 = _scatter_m(m, row, zero128)
            d0, d1 = _scatter_xc(dc, row, dcc, crow, zero128)
            hcur, x16, v16 = _upd_full(
                hcur, s0[:N], s1[:N], d0[:N, :16], d1[:N, :16], x16, v16, vs,
                p["Wh1"][0:HF], p["Wh1"][HF:2 * HF], p["bh1"].reshape(1, HF),
                p["Wh2"], p["bh2"].reshape(1, HF))
    return x16[:, :3], v16[:, :3]
